# baseline (speedup 1.0000x reference)
import functools

import jax
import jax.numpy as jnp
from jax import lax
from jax.experimental import pallas as pl
from jax.experimental.pallas import tpu as pltpu
from jax.experimental.pallas import tpu_sc as plsc

N = 10000
E = 320000
D = 128
BN_EPS = 1e-5

NC = 2
NS = 16
NW = NC * NS
EPW = E // NW
L = 16

CHD = 100
NCHD = EPW // CHD
RPT = 624
TAIL = N - NS * RPT
TOFF = NS * RPT

NBIN = 2
HB = N // NBIN
DUM = 8
ACCR = HB + DUM
CB = 128
NCB = 44
CAP = NCB * CB
ZR = 104
RPA = 312
TAILA = ACCR - NS * RPA
TOFFA = NS * RPA


def _deg_body(dst_hbm, ones_hbm, zeros1_hbm, deg0, deg1, dst_v, ones_v, dacc):
    cid = lax.axis_index("c")
    sid = lax.axis_index("s")
    wid = cid * NS + sid
    pltpu.sync_copy(dst_hbm.at[wid], dst_v)
    pltpu.sync_copy(ones_hbm, ones_v)
    pltpu.sync_copy(zeros1_hbm, dacc.at[pl.ds(sid * RPT, RPT)])

    @pl.when(sid == NS - 1)
    def _():
        pltpu.sync_copy(zeros1_hbm.at[pl.ds(0, TAIL)],
                        dacc.at[pl.ds(TOFF, TAIL)])

    plsc.subcore_barrier()

    def step(j, carry):
        pltpu.sync_copy(ones_v, dacc.at[dst_v.at[j]], add=True)
        return carry

    lax.fori_loop(0, NCHD, step, 0)
    plsc.subcore_barrier()
    sl = pl.ds(sid * RPT, RPT)
    tl = pl.ds(TOFF, TAIL)

    @pl.when(cid == 0)
    def _():
        pltpu.sync_copy(dacc.at[sl], deg0.at[sl])

        @pl.when(sid == NS - 1)
        def _():
            pltpu.sync_copy(dacc.at[tl], deg0.at[tl])

    @pl.when(cid == 1)
    def _():
        pltpu.sync_copy(dacc.at[sl], deg1.at[sl])

        @pl.when(sid == NS - 1)
        def _():
            pltpu.sync_copy(dacc.at[tl], deg1.at[tl])


def _bin_body(src_hbm, dst_hbm, oAs, oAd, oBs, oBd,
              stage_s, stage_d, bAs, bAd, bBs, bBd):
    cid = lax.axis_index("c")
    sid = lax.axis_index("s")
    wid = cid * NS + sid
    pltpu.sync_copy(src_hbm.at[pl.ds(wid * EPW, EPW)], stage_s)
    pltpu.sync_copy(dst_hbm.at[pl.ds(wid * EPW, EPW)], stage_d)

    iota = lax.iota(jnp.int32, L)
    pad_d = HB + (iota % DUM)
    pad_s = wid * 256 + iota * 16

    def prefill(q, carry):
        sl = pl.ds(q * L, L)
        bAs[sl] = pad_s
        bAd[sl] = pad_d
        bBs[sl] = pad_s
        bBd[sl] = pad_d
        return carry

    lax.fori_loop(0, CAP // L, prefill, 0)

    def step(i, carry):
        pA, pB = carry
        sl = pl.ds(i * L, L)
        s = stage_s[sl]
        d = stage_d[sl]
        m0 = d < HB
        n0 = plsc.all_reduce_population_count(m0)
        c = plsc.cumsum(m0.astype(jnp.int32))
        pos0 = pA + c - 1
        plsc.store_scatter(bAs, [pos0], s, mask=m0)
        plsc.store_scatter(bAd, [pos0], d, mask=m0)
        m1 = jnp.logical_not(m0)
        c1 = plsc.cumsum(m1.astype(jnp.int32))
        pos1 = pB + c1 - 1
        plsc.store_scatter(bBs, [pos1], s, mask=m1)
        plsc.store_scatter(bBd, [pos1], d - HB, mask=m1)
        return pA + n0, pB + (L - n0)

    lax.fori_loop(0, EPW // L, step,
                  (jnp.zeros((L,), jnp.int32), jnp.zeros((L,), jnp.int32)))

    out = pl.ds(wid * CAP, CAP)
    pltpu.sync_copy(bAs, oAs.at[out])
    pltpu.sync_copy(bAd, oAd.at[out])
    pltpu.sync_copy(bBs, oBs.at[out])
    pltpu.sync_copy(bBd, oBd.at[out])


def _agg_body(t_hbm, srcA, dstA, srcB, dstB, zeros_hbm, out0, out1,
              src_v, dst_v, r0, r1, r2, r3, acc,
              g0, g1, g2, g3, s0, s1, s2, s3):
    cid = lax.axis_index("c")
    sid = lax.axis_index("s")
    wid = cid * NS + sid
    bufs = (r0, r1, r2, r3)
    gsem = (g0, g1, g2, g3)
    ssem = (s0, s1, s2, s3)
    NBUF = 4
    ROUNDS = NCB // NBUF

    for k, (src_h, dst_h) in enumerate(((srcA, dstA), (srcB, dstB))):
        pltpu.sync_copy(src_h.at[wid], src_v)
        pltpu.sync_copy(dst_h.at[wid], dst_v)

        for q in range(RPA // ZR):
            pltpu.sync_copy(zeros_hbm,
                            acc.at[pl.ds(sid * RPA + q * ZR, ZR)])

        @pl.when(sid == NS - 1)
        def _():
            pltpu.sync_copy(zeros_hbm.at[pl.ds(0, TAILA)],
                            acc.at[pl.ds(TOFFA, TAILA)])

        plsc.subcore_barrier()

        for q in range(NBUF):
            pltpu.async_copy(t_hbm.at[src_v.at[q]], bufs[q], gsem[q])

        def rnd(jj, carry):
            j0 = jj * NBUF
            for q in range(NBUF):
                pltpu.make_async_copy(
                    t_hbm.at[src_v.at[j0 + q]], bufs[q], gsem[q]).wait()
                pltpu.async_copy(bufs[q], acc.at[dst_v.at[j0 + q]], ssem[q],
                                 add=True)
            for q in range(NBUF):
                pltpu.make_async_copy(
                    bufs[q], acc.at[dst_v.at[j0 + q]], ssem[q]).wait()

                @pl.when(j0 + NBUF + q < NCB)
                def _():
                    pltpu.async_copy(
                        t_hbm.at[src_v.at[j0 + NBUF + q]], bufs[q], gsem[q])
            return carry

        lax.fori_loop(0, ROUNDS, rnd, 0)

        plsc.subcore_barrier()
        sl = pl.ds(sid * RPA, RPA)
        osl = pl.ds(k * HB + sid * RPA, RPA)
        tsl = pl.ds(TOFFA, HB - TOFFA)
        otsl = pl.ds(k * HB + TOFFA, HB - TOFFA)

        @pl.when(cid == 0)
        def _():
            pltpu.sync_copy(acc.at[sl], out0.at[osl])

            @pl.when(sid == NS - 1)
            def _():
                pltpu.sync_copy(acc.at[tsl], out0.at[otsl])

        @pl.when(cid == 1)
        def _():
            pltpu.sync_copy(acc.at[sl], out1.at[osl])

            @pl.when(sid == NS - 1)
            def _():
                pltpu.sync_copy(acc.at[tsl], out1.at[otsl])

        plsc.subcore_barrier()


@functools.lru_cache(maxsize=None)
def _sc_kernels():
    mesh = plsc.VectorSubcoreMesh(
        core_axis_name="c", subcore_axis_name="s", num_cores=NC,
        num_subcores=NS)
    deg = functools.partial(
        pl.kernel,
        out_type=(
            jax.ShapeDtypeStruct((N, 1), jnp.float32),
            jax.ShapeDtypeStruct((N, 1), jnp.float32),
        ),
        mesh=mesh,
        scratch_types=(
            pltpu.VMEM((NCHD, CHD), jnp.int32),
            pltpu.VMEM((CHD, 1), jnp.float32),
            pltpu.VMEM_SHARED((N, 1), jnp.float32),
        ),
    )(_deg_body)
    bink = functools.partial(
        pl.kernel,
        out_type=tuple(
            jax.ShapeDtypeStruct((NW * CAP,), jnp.int32) for _ in range(4)
        ),
        mesh=mesh,
        scratch_types=(
            pltpu.VMEM((EPW,), jnp.int32),
            pltpu.VMEM((EPW,), jnp.int32),
            pltpu.VMEM((CAP,), jnp.int32),
            pltpu.VMEM((CAP,), jnp.int32),
            pltpu.VMEM((CAP,), jnp.int32),
            pltpu.VMEM((CAP,), jnp.int32),
        ),
        compiler_params=pltpu.CompilerParams(needs_layout_passes=False),
    )(_bin_body)
    agg = functools.partial(
        pl.kernel,
        out_type=(
            jax.ShapeDtypeStruct((N, D), jnp.float32),
            jax.ShapeDtypeStruct((N, D), jnp.float32),
        ),
        mesh=mesh,
        scratch_types=(
            pltpu.VMEM((NCB, CB), jnp.int32),
            pltpu.VMEM((NCB, CB), jnp.int32),
            pltpu.VMEM((CB, D), jnp.float32),
            pltpu.VMEM((CB, D), jnp.float32),
            pltpu.VMEM((CB, D), jnp.float32),
            pltpu.VMEM((CB, D), jnp.float32),
            pltpu.VMEM_SHARED((ACCR, D), jnp.float32),
            pltpu.SemaphoreType.DMA,
            pltpu.SemaphoreType.DMA,
            pltpu.SemaphoreType.DMA,
            pltpu.SemaphoreType.DMA,
            pltpu.SemaphoreType.DMA,
            pltpu.SemaphoreType.DMA,
            pltpu.SemaphoreType.DMA,
            pltpu.SemaphoreType.DMA,
        ),
    )(_agg_body)
    return deg, bink, agg


BM = 1000
GRID = N // BM


def _dinv(d0, d1):
    deg = d0 + d1
    return jnp.where(deg > 0.0, lax.rsqrt(deg), 0.0)


def _lin_body(d0_ref, d1_ref, x_ref, w_ref, b_ref, o_ref):
    dinv = _dinv(d0_ref[...], d1_ref[...])
    h = jnp.dot(x_ref[...], w_ref[...], preferred_element_type=jnp.float32,
                precision=lax.Precision.HIGHEST)
    o_ref[...] = (h + b_ref[...]) * dinv


def _mid_body(d0_ref, d1_ref, a0_ref, a1_ref, g_ref, be_ref, w_ref, b_ref,
              o_ref):
    dinv = _dinv(d0_ref[...], d1_ref[...])
    z = (a0_ref[...] + a1_ref[...]) * dinv
    gs = g_ref[...] * lax.rsqrt(jnp.float32(1.0 + BN_EPS))
    z = jnp.maximum(z * gs + be_ref[...], 0.0)
    h = jnp.dot(z, w_ref[...], preferred_element_type=jnp.float32,
                precision=lax.Precision.HIGHEST)
    o_ref[...] = (h + b_ref[...]) * dinv


def _lsm_body(d0_ref, d1_ref, a0_ref, a1_ref, o_ref):
    dinv = _dinv(d0_ref[...], d1_ref[...])
    z = (a0_ref[...] + a1_ref[...]) * dinv
    m = jnp.max(z, axis=-1, keepdims=True)
    s = z - m
    o_ref[...] = s - jnp.log(jnp.sum(jnp.exp(s), axis=-1, keepdims=True))


_col = pl.BlockSpec((BM, 1), lambda i: (i, 0))
_row = pl.BlockSpec((BM, D), lambda i: (i, 0))
_vec = pl.BlockSpec((1, D), lambda i: (0, 0))
_mat = pl.BlockSpec((D, D), lambda i: (0, 0))
_out = jax.ShapeDtypeStruct((N, D), jnp.float32)

_lin = pl.pallas_call(
    _lin_body, grid=(GRID,),
    in_specs=[_col, _col, _row, _mat, _vec],
    out_specs=_row, out_shape=_out)

_mid = pl.pallas_call(
    _mid_body, grid=(GRID,),
    in_specs=[_col, _col, _row, _row, _vec, _vec, _mat, _vec],
    out_specs=_row, out_shape=_out)

_lsm = pl.pallas_call(
    _lsm_body, grid=(GRID,),
    in_specs=[_col, _col, _row, _row],
    out_specs=_row, out_shape=_out)


def kernel(x, adj_t, W1, b1, g1, be1, W2, b2, g2, be2, W3, b3):
    src = adj_t[0]
    dst = adj_t[1]
    dst3 = dst.reshape(NW, NCHD, CHD)
    ones_c = jnp.ones((CHD, 1), jnp.float32)
    zeros1 = jnp.zeros((RPT, 1), jnp.float32)
    zeros_t = jnp.zeros((ZR, D), jnp.float32)
    b1r, b2r, b3r = (v.reshape(1, D) for v in (b1, b2, b3))
    g1r, g2r = g1.reshape(1, D), g2.reshape(1, D)
    be1r, be2r = be1.reshape(1, D), be2.reshape(1, D)

    _deg, _bin, _agg = _sc_kernels()
    d0, d1 = _deg(dst3, ones_c, zeros1)
    sAf, dAf, sBf, dBf = _bin(src, dst)
    sA3 = sAf.reshape(NW, NCB, CB)
    dA3 = dAf.reshape(NW, NCB, CB)
    sB3 = sBf.reshape(NW, NCB, CB)
    dB3 = dBf.reshape(NW, NCB, CB)

    t1 = _lin(d0, d1, x, W1, b1r)
    a0, a1 = _agg(t1, sA3, dA3, sB3, dB3, zeros_t)
    t2 = _mid(d0, d1, a0, a1, g1r, be1r, W2, b2r)
    a0, a1 = _agg(t2, sA3, dA3, sB3, dB3, zeros_t)
    t3 = _mid(d0, d1, a0, a1, g2r, be2r, W3, b3r)
    a0, a1 = _agg(t3, sA3, dA3, sB3, dB3, zeros_t)
    return _lsm(d0, d1, a0, a1)


# speedup vs baseline: 1.1351x; 1.1351x over previous
import functools

import jax
import jax.numpy as jnp
from jax import lax
from jax.experimental import pallas as pl
from jax.experimental.pallas import tpu as pltpu
from jax.experimental.pallas import tpu_sc as plsc

N = 10000
E = 320000
D = 128
BN_EPS = 1e-5

NC = 2
NS = 16
NW = NC * NS
EPW = E // NW
L = 16

CHD = 100
NCHD = EPW // CHD
RPT = 624
TAIL = N - NS * RPT
TOFF = NS * RPT

NBIN = 2
HB = N // NBIN
DUM = 8
ACCR = HB + DUM
CB = 128
NCB = 42
CAP = NCB * CB
ZR = 104
RPA = 312
TAILA = ACCR - NS * RPA
TOFFA = NS * RPA


def _deg_body(dst_hbm, ones_hbm, zeros1_hbm, deg0, deg1, dst_v, ones_v, dacc):
    cid = lax.axis_index("c")
    sid = lax.axis_index("s")
    wid = cid * NS + sid
    pltpu.sync_copy(dst_hbm.at[wid], dst_v)
    pltpu.sync_copy(ones_hbm, ones_v)
    pltpu.sync_copy(zeros1_hbm, dacc.at[pl.ds(sid * RPT, RPT)])

    @pl.when(sid == NS - 1)
    def _():
        pltpu.sync_copy(zeros1_hbm.at[pl.ds(0, TAIL)],
                        dacc.at[pl.ds(TOFF, TAIL)])

    plsc.subcore_barrier()

    def step(j, carry):
        pltpu.sync_copy(ones_v, dacc.at[dst_v.at[j]], add=True)
        return carry

    lax.fori_loop(0, NCHD, step, 0)
    plsc.subcore_barrier()
    sl = pl.ds(sid * RPT, RPT)
    tl = pl.ds(TOFF, TAIL)

    @pl.when(cid == 0)
    def _():
        pltpu.sync_copy(dacc.at[sl], deg0.at[sl])

        @pl.when(sid == NS - 1)
        def _():
            pltpu.sync_copy(dacc.at[tl], deg0.at[tl])

    @pl.when(cid == 1)
    def _():
        pltpu.sync_copy(dacc.at[sl], deg1.at[sl])

        @pl.when(sid == NS - 1)
        def _():
            pltpu.sync_copy(dacc.at[tl], deg1.at[tl])


def _bin_body(src_hbm, dst_hbm, oAs, oAd, oBs, oBd,
              stage_s, stage_d, bAs, bAd, bBs, bBd):
    cid = lax.axis_index("c")
    sid = lax.axis_index("s")
    wid = cid * NS + sid
    pltpu.sync_copy(src_hbm.at[pl.ds(wid * EPW, EPW)], stage_s)
    pltpu.sync_copy(dst_hbm.at[pl.ds(wid * EPW, EPW)], stage_d)

    iota = lax.iota(jnp.int32, L)
    pad_d = HB + (iota % DUM)
    pad_s = wid * 256 + iota * 16

    def prefill(q, carry):
        sl = pl.ds(q * L, L)
        bAs[sl] = pad_s
        bAd[sl] = pad_d
        bBs[sl] = pad_s
        bBd[sl] = pad_d
        return carry

    lax.fori_loop(0, CAP // L, prefill, 0)

    def step(i, carry):
        pA, pB = carry
        sl = pl.ds(i * L, L)
        s = stage_s[sl]
        d = stage_d[sl]
        m0 = d < HB
        n0 = plsc.all_reduce_population_count(m0)
        c = plsc.cumsum(m0.astype(jnp.int32))
        pos0 = pA + c - 1
        plsc.store_scatter(bAs, [pos0], s, mask=m0)
        plsc.store_scatter(bAd, [pos0], d, mask=m0)
        m1 = jnp.logical_not(m0)
        c1 = plsc.cumsum(m1.astype(jnp.int32))
        pos1 = pB + c1 - 1
        plsc.store_scatter(bBs, [pos1], s, mask=m1)
        plsc.store_scatter(bBd, [pos1], d - HB, mask=m1)
        return pA + n0, pB + (L - n0)

    lax.fori_loop(0, EPW // L, step,
                  (jnp.zeros((L,), jnp.int32), jnp.zeros((L,), jnp.int32)))

    out = pl.ds(wid * CAP, CAP)
    pltpu.sync_copy(bAs, oAs.at[out])
    pltpu.sync_copy(bAd, oAd.at[out])
    pltpu.sync_copy(bBs, oBs.at[out])
    pltpu.sync_copy(bBd, oBd.at[out])


def _agg_body(t_hbm, srcA, dstA, srcB, dstB, zeros_hbm, out,
              src_v, dst_v, r0, r1, r2, r3, acc,
              g0, g1, g2, g3, s0, s1, s2, s3):
    cid = lax.axis_index("c")
    sid = lax.axis_index("s")
    bufs = (r0, r1, r2, r3)
    gsem = (g0, g1, g2, g3)
    ssem = (s0, s1, s2, s3)
    NBUF = 4
    NC2 = 2 * NCB
    ROUNDS = NC2 // NBUF

    @pl.when(cid == 0)
    def _():
        pltpu.sync_copy(srcA.at[2 * sid], src_v.at[pl.ds(0, NCB)])
        pltpu.sync_copy(srcA.at[2 * sid + 1], src_v.at[pl.ds(NCB, NCB)])
        pltpu.sync_copy(dstA.at[2 * sid], dst_v.at[pl.ds(0, NCB)])
        pltpu.sync_copy(dstA.at[2 * sid + 1], dst_v.at[pl.ds(NCB, NCB)])

    @pl.when(cid == 1)
    def _():
        pltpu.sync_copy(srcB.at[2 * sid], src_v.at[pl.ds(0, NCB)])
        pltpu.sync_copy(srcB.at[2 * sid + 1], src_v.at[pl.ds(NCB, NCB)])
        pltpu.sync_copy(dstB.at[2 * sid], dst_v.at[pl.ds(0, NCB)])
        pltpu.sync_copy(dstB.at[2 * sid + 1], dst_v.at[pl.ds(NCB, NCB)])

    for q in range(RPA // ZR):
        pltpu.sync_copy(zeros_hbm, acc.at[pl.ds(sid * RPA + q * ZR, ZR)])

    @pl.when(sid == NS - 1)
    def _():
        pltpu.sync_copy(zeros_hbm.at[pl.ds(0, TAILA)],
                        acc.at[pl.ds(TOFFA, TAILA)])

    plsc.subcore_barrier()

    for q in range(NBUF):
        pltpu.async_copy(t_hbm.at[src_v.at[q]], bufs[q], gsem[q])

    def rnd(jj, carry):
        j0 = jj * NBUF
        for q in range(NBUF):
            pltpu.make_async_copy(
                t_hbm.at[src_v.at[j0 + q]], bufs[q], gsem[q]).wait()
            pltpu.async_copy(bufs[q], acc.at[dst_v.at[j0 + q]], ssem[q],
                             add=True)
        for q in range(NBUF):
            pltpu.make_async_copy(
                bufs[q], acc.at[dst_v.at[j0 + q]], ssem[q]).wait()

            @pl.when(j0 + NBUF + q < NC2)
            def _():
                pltpu.async_copy(
                    t_hbm.at[src_v.at[j0 + NBUF + q]], bufs[q], gsem[q])
        return carry

    lax.fori_loop(0, ROUNDS, rnd, 0)

    plsc.subcore_barrier()
    sl = pl.ds(sid * RPA, RPA)
    tsl = pl.ds(TOFFA, HB - TOFFA)

    @pl.when(cid == 0)
    def _():
        pltpu.sync_copy(acc.at[sl], out.at[pl.ds(sid * RPA, RPA)])

        @pl.when(sid == NS - 1)
        def _():
            pltpu.sync_copy(acc.at[tsl], out.at[pl.ds(TOFFA, HB - TOFFA)])

    @pl.when(cid == 1)
    def _():
        pltpu.sync_copy(acc.at[sl], out.at[pl.ds(HB + sid * RPA, RPA)])

        @pl.when(sid == NS - 1)
        def _():
            pltpu.sync_copy(acc.at[tsl],
                            out.at[pl.ds(HB + TOFFA, HB - TOFFA)])


@functools.lru_cache(maxsize=None)
def _sc_kernels():
    mesh = plsc.VectorSubcoreMesh(
        core_axis_name="c", subcore_axis_name="s", num_cores=NC,
        num_subcores=NS)
    deg = functools.partial(
        pl.kernel,
        out_type=(
            jax.ShapeDtypeStruct((N, 1), jnp.float32),
            jax.ShapeDtypeStruct((N, 1), jnp.float32),
        ),
        mesh=mesh,
        scratch_types=(
            pltpu.VMEM((NCHD, CHD), jnp.int32),
            pltpu.VMEM((CHD, 1), jnp.float32),
            pltpu.VMEM_SHARED((N, 1), jnp.float32),
        ),
    )(_deg_body)
    bink = functools.partial(
        pl.kernel,
        out_type=tuple(
            jax.ShapeDtypeStruct((NW * CAP,), jnp.int32) for _ in range(4)
        ),
        mesh=mesh,
        scratch_types=(
            pltpu.VMEM((EPW,), jnp.int32),
            pltpu.VMEM((EPW,), jnp.int32),
            pltpu.VMEM((CAP,), jnp.int32),
            pltpu.VMEM((CAP,), jnp.int32),
            pltpu.VMEM((CAP,), jnp.int32),
            pltpu.VMEM((CAP,), jnp.int32),
        ),
        compiler_params=pltpu.CompilerParams(needs_layout_passes=False),
    )(_bin_body)
    agg = functools.partial(
        pl.kernel,
        out_type=jax.ShapeDtypeStruct((N, D), jnp.float32),
        mesh=mesh,
        scratch_types=(
            pltpu.VMEM((2 * NCB, CB), jnp.int32),
            pltpu.VMEM((2 * NCB, CB), jnp.int32),
            pltpu.VMEM((CB, D), jnp.float32),
            pltpu.VMEM((CB, D), jnp.float32),
            pltpu.VMEM((CB, D), jnp.float32),
            pltpu.VMEM((CB, D), jnp.float32),
            pltpu.VMEM_SHARED((ACCR, D), jnp.float32),
            pltpu.SemaphoreType.DMA,
            pltpu.SemaphoreType.DMA,
            pltpu.SemaphoreType.DMA,
            pltpu.SemaphoreType.DMA,
            pltpu.SemaphoreType.DMA,
            pltpu.SemaphoreType.DMA,
            pltpu.SemaphoreType.DMA,
            pltpu.SemaphoreType.DMA,
        ),
    )(_agg_body)
    return deg, bink, agg


BM = 1000
GRID = N // BM


def _dinv(d0, d1):
    deg = d0 + d1
    return jnp.where(deg > 0.0, lax.rsqrt(deg), 0.0)


def _lin_body(d0_ref, d1_ref, x_ref, w_ref, b_ref, o_ref):
    dinv = _dinv(d0_ref[...], d1_ref[...])
    h = jnp.dot(x_ref[...], w_ref[...], preferred_element_type=jnp.float32,
                precision=lax.Precision.HIGHEST)
    o_ref[...] = (h + b_ref[...]) * dinv


def _mid_body(d0_ref, d1_ref, a_ref, g_ref, be_ref, w_ref, b_ref, o_ref):
    dinv = _dinv(d0_ref[...], d1_ref[...])
    z = a_ref[...] * dinv
    gs = g_ref[...] * lax.rsqrt(jnp.float32(1.0 + BN_EPS))
    z = jnp.maximum(z * gs + be_ref[...], 0.0)
    h = jnp.dot(z, w_ref[...], preferred_element_type=jnp.float32,
                precision=lax.Precision.HIGHEST)
    o_ref[...] = (h + b_ref[...]) * dinv


def _lsm_body(d0_ref, d1_ref, a_ref, o_ref):
    dinv = _dinv(d0_ref[...], d1_ref[...])
    z = a_ref[...] * dinv
    m = jnp.max(z, axis=-1, keepdims=True)
    s = z - m
    o_ref[...] = s - jnp.log(jnp.sum(jnp.exp(s), axis=-1, keepdims=True))


_col = pl.BlockSpec((BM, 1), lambda i: (i, 0))
_row = pl.BlockSpec((BM, D), lambda i: (i, 0))
_vec = pl.BlockSpec((1, D), lambda i: (0, 0))
_mat = pl.BlockSpec((D, D), lambda i: (0, 0))
_out = jax.ShapeDtypeStruct((N, D), jnp.float32)

_lin = pl.pallas_call(
    _lin_body, grid=(GRID,),
    in_specs=[_col, _col, _row, _mat, _vec],
    out_specs=_row, out_shape=_out)

_mid = pl.pallas_call(
    _mid_body, grid=(GRID,),
    in_specs=[_col, _col, _row, _vec, _vec, _mat, _vec],
    out_specs=_row, out_shape=_out)

_lsm = pl.pallas_call(
    _lsm_body, grid=(GRID,),
    in_specs=[_col, _col, _row],
    out_specs=_row, out_shape=_out)


def kernel(x, adj_t, W1, b1, g1, be1, W2, b2, g2, be2, W3, b3):
    src = adj_t[0]
    dst = adj_t[1]
    dst3 = dst.reshape(NW, NCHD, CHD)
    ones_c = jnp.ones((CHD, 1), jnp.float32)
    zeros1 = jnp.zeros((RPT, 1), jnp.float32)
    zeros_t = jnp.zeros((ZR, D), jnp.float32)
    b1r, b2r, b3r = (v.reshape(1, D) for v in (b1, b2, b3))
    g1r, g2r = g1.reshape(1, D), g2.reshape(1, D)
    be1r, be2r = be1.reshape(1, D), be2.reshape(1, D)

    _deg, _bin, _agg = _sc_kernels()
    d0, d1 = _deg(dst3, ones_c, zeros1)
    sAf, dAf, sBf, dBf = _bin(src, dst)
    sA3 = sAf.reshape(NW, NCB, CB)
    dA3 = dAf.reshape(NW, NCB, CB)
    sB3 = sBf.reshape(NW, NCB, CB)
    dB3 = dBf.reshape(NW, NCB, CB)

    t1 = _lin(d0, d1, x, W1, b1r)
    a = _agg(t1, sA3, dA3, sB3, dB3, zeros_t)
    t2 = _mid(d0, d1, a, g1r, be1r, W2, b2r)
    a = _agg(t2, sA3, dA3, sB3, dB3, zeros_t)
    t3 = _mid(d0, d1, a, g2r, be2r, W3, b3r)
    a = _agg(t3, sA3, dA3, sB3, dB3, zeros_t)
    return _lsm(d0, d1, a)


# speedup vs baseline: 1.1841x; 1.0432x over previous
import functools

import jax
import jax.numpy as jnp
from jax import lax
from jax.experimental import pallas as pl
from jax.experimental.pallas import tpu as pltpu
from jax.experimental.pallas import tpu_sc as plsc

N = 10000
E = 320000
D = 128
BN_EPS = 1e-5

NC = 2
NS = 16
NW = NC * NS
EPW = E // NW
L = 16

CHD = 100
NCHD = EPW // CHD
RPT = 624
TAIL = N - NS * RPT
TOFF = NS * RPT

NBIN = 2
HB = N // NBIN
DUM = 8
ACCR = HB + DUM
CB = 128
NCB = 42
CAP = NCB * CB
ZR = 104
RPA = 312
TAILA = ACCR - NS * RPA
TOFFA = NS * RPA


def _deg_body(dst_hbm, ones_hbm, zeros1_hbm, deg0, deg1, dst_v, ones_v, dacc):
    cid = lax.axis_index("c")
    sid = lax.axis_index("s")
    wid = cid * NS + sid
    pltpu.sync_copy(dst_hbm.at[wid], dst_v)
    pltpu.sync_copy(ones_hbm, ones_v)
    pltpu.sync_copy(zeros1_hbm, dacc.at[pl.ds(sid * RPT, RPT)])

    @pl.when(sid == NS - 1)
    def _():
        pltpu.sync_copy(zeros1_hbm.at[pl.ds(0, TAIL)],
                        dacc.at[pl.ds(TOFF, TAIL)])

    plsc.subcore_barrier()

    def step(j, carry):
        pltpu.sync_copy(ones_v, dacc.at[dst_v.at[j]], add=True)
        return carry

    lax.fori_loop(0, NCHD, step, 0)
    plsc.subcore_barrier()
    sl = pl.ds(sid * RPT, RPT)
    tl = pl.ds(TOFF, TAIL)

    @pl.when(cid == 0)
    def _():
        pltpu.sync_copy(dacc.at[sl], deg0.at[sl])

        @pl.when(sid == NS - 1)
        def _():
            pltpu.sync_copy(dacc.at[tl], deg0.at[tl])

    @pl.when(cid == 1)
    def _():
        pltpu.sync_copy(dacc.at[sl], deg1.at[sl])

        @pl.when(sid == NS - 1)
        def _():
            pltpu.sync_copy(dacc.at[tl], deg1.at[tl])


def _bin_body(src_hbm, dst_hbm, oAs, oAd, oBs, oBd,
              stage_s, stage_d, bAs, bAd, bBs, bBd):
    cid = lax.axis_index("c")
    sid = lax.axis_index("s")
    wid = cid * NS + sid
    pltpu.sync_copy(src_hbm.at[pl.ds(wid * EPW, EPW)], stage_s)
    pltpu.sync_copy(dst_hbm.at[pl.ds(wid * EPW, EPW)], stage_d)

    iota = lax.iota(jnp.int32, L)
    pad_d = HB + (iota % DUM)
    pad_s = wid * 256 + iota * 16

    def prefill(q, carry):
        sl = pl.ds(q * L, L)
        bAs[sl] = pad_s
        bAd[sl] = pad_d
        bBs[sl] = pad_s
        bBd[sl] = pad_d
        return carry

    lax.fori_loop(0, CAP // L, prefill, 0)

    def step(i, carry):
        pA, pB = carry
        sl = pl.ds(i * L, L)
        s = stage_s[sl]
        d = stage_d[sl]
        m0 = d < HB
        n0 = plsc.all_reduce_population_count(m0)
        c = plsc.cumsum(m0.astype(jnp.int32))
        pos0 = pA + c - 1
        plsc.store_scatter(bAs, [pos0], s, mask=m0)
        plsc.store_scatter(bAd, [pos0], d, mask=m0)
        m1 = jnp.logical_not(m0)
        c1 = plsc.cumsum(m1.astype(jnp.int32))
        pos1 = pB + c1 - 1
        plsc.store_scatter(bBs, [pos1], s, mask=m1)
        plsc.store_scatter(bBd, [pos1], d - HB, mask=m1)
        return pA + n0, pB + (L - n0)

    lax.fori_loop(0, EPW // L, step,
                  (jnp.zeros((L,), jnp.int32), jnp.zeros((L,), jnp.int32)))

    out = pl.ds(wid * CAP, CAP)
    pltpu.sync_copy(bAs, oAs.at[out])
    pltpu.sync_copy(bAd, oAd.at[out])
    pltpu.sync_copy(bBs, oBs.at[out])
    pltpu.sync_copy(bBd, oBd.at[out])


def _agg_body(t_hbm, srcA, dstA, srcB, dstB, zeros_hbm, out,
              src_v, dst_v, r0, r1, r2, r3, acc,
              g0, g1, g2, g3, s0, s1, s2, s3):
    cid = lax.axis_index("c")
    sid = lax.axis_index("s")
    bufs = (r0, r1, r2, r3)
    gsem = (g0, g1, g2, g3)
    ssem = (s0, s1, s2, s3)
    NBUF = 4
    NC2 = 2 * NCB
    ROUNDS = NC2 // NBUF

    @pl.when(cid == 0)
    def _():
        pltpu.sync_copy(srcA.at[2 * sid], src_v.at[pl.ds(0, NCB)])
        pltpu.sync_copy(srcA.at[2 * sid + 1], src_v.at[pl.ds(NCB, NCB)])
        pltpu.sync_copy(dstA.at[2 * sid], dst_v.at[pl.ds(0, NCB)])
        pltpu.sync_copy(dstA.at[2 * sid + 1], dst_v.at[pl.ds(NCB, NCB)])

    @pl.when(cid == 1)
    def _():
        pltpu.sync_copy(srcB.at[2 * sid], src_v.at[pl.ds(0, NCB)])
        pltpu.sync_copy(srcB.at[2 * sid + 1], src_v.at[pl.ds(NCB, NCB)])
        pltpu.sync_copy(dstB.at[2 * sid], dst_v.at[pl.ds(0, NCB)])
        pltpu.sync_copy(dstB.at[2 * sid + 1], dst_v.at[pl.ds(NCB, NCB)])

    for q in range(RPA // ZR):
        pltpu.sync_copy(zeros_hbm, acc.at[pl.ds(sid * RPA + q * ZR, ZR)])

    @pl.when(sid == NS - 1)
    def _():
        pltpu.sync_copy(zeros_hbm.at[pl.ds(0, TAILA)],
                        acc.at[pl.ds(TOFFA, TAILA)])

    plsc.subcore_barrier()

    for q in range(NBUF):
        pltpu.async_copy(t_hbm.at[src_v.at[q]], bufs[q], gsem[q])

    def rnd(jj, carry):
        j0 = jj * NBUF
        for q in range(NBUF):
            pltpu.make_async_copy(
                t_hbm.at[src_v.at[j0 + q]], bufs[q], gsem[q]).wait()
            pltpu.async_copy(bufs[q], acc.at[dst_v.at[j0 + q]], ssem[q],
                             add=True)
        for q in range(NBUF):
            pltpu.make_async_copy(
                bufs[q], acc.at[dst_v.at[j0 + q]], ssem[q]).wait()

            @pl.when(j0 + NBUF + q < NC2)
            def _():
                pltpu.async_copy(
                    t_hbm.at[src_v.at[j0 + NBUF + q]], bufs[q], gsem[q])
        return carry

    lax.fori_loop(0, ROUNDS, rnd, 0)

    plsc.subcore_barrier()
    sl = pl.ds(sid * RPA, RPA)
    tsl = pl.ds(TOFFA, HB - TOFFA)

    @pl.when(cid == 0)
    def _():
        pltpu.sync_copy(acc.at[sl], out.at[pl.ds(sid * RPA, RPA)])

        @pl.when(sid == NS - 1)
        def _():
            pltpu.sync_copy(acc.at[tsl], out.at[pl.ds(TOFFA, HB - TOFFA)])

    @pl.when(cid == 1)
    def _():
        pltpu.sync_copy(acc.at[sl], out.at[pl.ds(HB + sid * RPA, RPA)])

        @pl.when(sid == NS - 1)
        def _():
            pltpu.sync_copy(acc.at[tsl],
                            out.at[pl.ds(HB + TOFFA, HB - TOFFA)])


@functools.lru_cache(maxsize=None)
def _sc_kernels():
    mesh = plsc.VectorSubcoreMesh(
        core_axis_name="c", subcore_axis_name="s", num_cores=NC,
        num_subcores=NS)
    deg = functools.partial(
        pl.kernel,
        out_type=(
            jax.ShapeDtypeStruct((N, 1), jnp.float32),
            jax.ShapeDtypeStruct((N, 1), jnp.float32),
        ),
        mesh=mesh,
        scratch_types=(
            pltpu.VMEM((NCHD, CHD), jnp.int32),
            pltpu.VMEM((CHD, 1), jnp.float32),
            pltpu.VMEM_SHARED((N, 1), jnp.float32),
        ),
    )(_deg_body)
    bink = functools.partial(
        pl.kernel,
        out_type=tuple(
            jax.ShapeDtypeStruct((NW * CAP,), jnp.int32) for _ in range(4)
        ),
        mesh=mesh,
        scratch_types=(
            pltpu.VMEM((EPW,), jnp.int32),
            pltpu.VMEM((EPW,), jnp.int32),
            pltpu.VMEM((CAP,), jnp.int32),
            pltpu.VMEM((CAP,), jnp.int32),
            pltpu.VMEM((CAP,), jnp.int32),
            pltpu.VMEM((CAP,), jnp.int32),
        ),
        compiler_params=pltpu.CompilerParams(needs_layout_passes=False),
    )(_bin_body)
    agg = functools.partial(
        pl.kernel,
        out_type=jax.ShapeDtypeStruct((N, D), jnp.float32),
        mesh=mesh,
        scratch_types=(
            pltpu.VMEM((2 * NCB, CB), jnp.int32),
            pltpu.VMEM((2 * NCB, CB), jnp.int32),
            pltpu.VMEM((CB, D), jnp.float32),
            pltpu.VMEM((CB, D), jnp.float32),
            pltpu.VMEM((CB, D), jnp.float32),
            pltpu.VMEM((CB, D), jnp.float32),
            pltpu.VMEM_SHARED((ACCR, D), jnp.float32),
            pltpu.SemaphoreType.DMA,
            pltpu.SemaphoreType.DMA,
            pltpu.SemaphoreType.DMA,
            pltpu.SemaphoreType.DMA,
            pltpu.SemaphoreType.DMA,
            pltpu.SemaphoreType.DMA,
            pltpu.SemaphoreType.DMA,
            pltpu.SemaphoreType.DMA,
        ),
    )(_agg_body)
    return deg, bink, agg


BM = 2000
GRID = N // BM


def _dinv(d0, d1):
    deg = d0 + d1
    return jnp.where(deg > 0.0, lax.rsqrt(deg), 0.0)


def _lin_body(d0_ref, d1_ref, x_ref, w_ref, b_ref, o_ref):
    dinv = _dinv(d0_ref[...], d1_ref[...])
    h = jnp.dot(x_ref[...], w_ref[...], preferred_element_type=jnp.float32)
    o_ref[...] = (h + b_ref[...]) * dinv


def _mid_body(d0_ref, d1_ref, a_ref, g_ref, be_ref, w_ref, b_ref, o_ref):
    dinv = _dinv(d0_ref[...], d1_ref[...])
    z = a_ref[...] * dinv
    gs = g_ref[...] * lax.rsqrt(jnp.float32(1.0 + BN_EPS))
    z = jnp.maximum(z * gs + be_ref[...], 0.0)
    h = jnp.dot(z, w_ref[...], preferred_element_type=jnp.float32)
    o_ref[...] = (h + b_ref[...]) * dinv


def _lsm_body(d0_ref, d1_ref, a_ref, o_ref):
    dinv = _dinv(d0_ref[...], d1_ref[...])
    z = a_ref[...] * dinv
    m = jnp.max(z, axis=-1, keepdims=True)
    s = z - m
    o_ref[...] = s - jnp.log(jnp.sum(jnp.exp(s), axis=-1, keepdims=True))


_col = pl.BlockSpec((BM, 1), lambda i: (i, 0))
_row = pl.BlockSpec((BM, D), lambda i: (i, 0))
_vec = pl.BlockSpec((1, D), lambda i: (0, 0))
_mat = pl.BlockSpec((D, D), lambda i: (0, 0))
_out = jax.ShapeDtypeStruct((N, D), jnp.float32)

_lin = pl.pallas_call(
    _lin_body, grid=(GRID,),
    in_specs=[_col, _col, _row, _mat, _vec],
    out_specs=_row, out_shape=_out)

_mid = pl.pallas_call(
    _mid_body, grid=(GRID,),
    in_specs=[_col, _col, _row, _vec, _vec, _mat, _vec],
    out_specs=_row, out_shape=_out)

_lsm = pl.pallas_call(
    _lsm_body, grid=(GRID,),
    in_specs=[_col, _col, _row],
    out_specs=_row, out_shape=_out)


def kernel(x, adj_t, W1, b1, g1, be1, W2, b2, g2, be2, W3, b3):
    src = adj_t[0]
    dst = adj_t[1]
    dst3 = dst.reshape(NW, NCHD, CHD)
    ones_c = jnp.ones((CHD, 1), jnp.float32)
    zeros1 = jnp.zeros((RPT, 1), jnp.float32)
    zeros_t = jnp.zeros((ZR, D), jnp.float32)
    b1r, b2r, b3r = (v.reshape(1, D) for v in (b1, b2, b3))
    g1r, g2r = g1.reshape(1, D), g2.reshape(1, D)
    be1r, be2r = be1.reshape(1, D), be2.reshape(1, D)

    _deg, _bin, _agg = _sc_kernels()
    d0, d1 = _deg(dst3, ones_c, zeros1)
    sAf, dAf, sBf, dBf = _bin(src, dst)
    sA3 = sAf.reshape(NW, NCB, CB)
    dA3 = dAf.reshape(NW, NCB, CB)
    sB3 = sBf.reshape(NW, NCB, CB)
    dB3 = dBf.reshape(NW, NCB, CB)

    t1 = _lin(d0, d1, x, W1, b1r)
    a = _agg(t1, sA3, dA3, sB3, dB3, zeros_t)
    t2 = _mid(d0, d1, a, g1r, be1r, W2, b2r)
    a = _agg(t2, sA3, dA3, sB3, dB3, zeros_t)
    t3 = _mid(d0, d1, a, g2r, be2r, W3, b3r)
    a = _agg(t3, sA3, dA3, sB3, dB3, zeros_t)
    return _lsm(d0, d1, a)


# speedup vs baseline: 1.1880x; 1.0033x over previous
import functools

import jax
import jax.numpy as jnp
from jax import lax
from jax.experimental import pallas as pl
from jax.experimental.pallas import tpu as pltpu
from jax.experimental.pallas import tpu_sc as plsc

N = 10000
E = 320000
D = 128
BN_EPS = 1e-5

NC = 2
NS = 16
NW = NC * NS
EPW = E // NW
L = 16

CHD = 125
NCHD = EPW // CHD
RPT = 624
TAIL = N - NS * RPT
TOFF = NS * RPT

NBIN = 2
HB = N // NBIN
DUM = 8
ACCR = HB + DUM
CB = 128
NCB = 42
CAP = NCB * CB
ZR = 104
RPA = 312
TAILA = ACCR - NS * RPA
TOFFA = NS * RPA


def _deg_body(dst_hbm, ones_hbm, zeros1_hbm, deg0, deg1, dst_v, ones_v, dacc):
    cid = lax.axis_index("c")
    sid = lax.axis_index("s")
    wid = cid * NS + sid
    pltpu.sync_copy(dst_hbm.at[wid], dst_v)
    pltpu.sync_copy(ones_hbm, ones_v)
    pltpu.sync_copy(zeros1_hbm, dacc.at[pl.ds(sid * RPT, RPT)])

    @pl.when(sid == NS - 1)
    def _():
        pltpu.sync_copy(zeros1_hbm.at[pl.ds(0, TAIL)],
                        dacc.at[pl.ds(TOFF, TAIL)])

    plsc.subcore_barrier()

    def step(j, carry):
        pltpu.sync_copy(ones_v, dacc.at[dst_v.at[j]], add=True)
        return carry

    lax.fori_loop(0, NCHD, step, 0)
    plsc.subcore_barrier()
    sl = pl.ds(sid * RPT, RPT)
    tl = pl.ds(TOFF, TAIL)

    @pl.when(cid == 0)
    def _():
        pltpu.sync_copy(dacc.at[sl], deg0.at[sl])

        @pl.when(sid == NS - 1)
        def _():
            pltpu.sync_copy(dacc.at[tl], deg0.at[tl])

    @pl.when(cid == 1)
    def _():
        pltpu.sync_copy(dacc.at[sl], deg1.at[sl])

        @pl.when(sid == NS - 1)
        def _():
            pltpu.sync_copy(dacc.at[tl], deg1.at[tl])


def _bin_body(src_hbm, dst_hbm, oAs, oAd, oBs, oBd,
              stage_s, stage_d, bAs, bAd, bBs, bBd):
    cid = lax.axis_index("c")
    sid = lax.axis_index("s")
    wid = cid * NS + sid
    pltpu.sync_copy(src_hbm.at[pl.ds(wid * EPW, EPW)], stage_s)
    pltpu.sync_copy(dst_hbm.at[pl.ds(wid * EPW, EPW)], stage_d)

    iota = lax.iota(jnp.int32, L)
    pad_d = HB + (iota % DUM)
    pad_s = wid * 256 + iota * 16

    def prefill(q, carry):
        sl = pl.ds(q * L, L)
        bAs[sl] = pad_s
        bAd[sl] = pad_d
        bBs[sl] = pad_s
        bBd[sl] = pad_d
        return carry

    lax.fori_loop(0, CAP // L, prefill, 0)

    def step(i, carry):
        pA, pB = carry
        sl = pl.ds(i * L, L)
        s = stage_s[sl]
        d = stage_d[sl]
        m0 = d < HB
        n0 = plsc.all_reduce_population_count(m0)
        c = plsc.cumsum(m0.astype(jnp.int32))
        pos0 = pA + c - 1
        plsc.store_scatter(bAs, [pos0], s, mask=m0)
        plsc.store_scatter(bAd, [pos0], d, mask=m0)
        m1 = jnp.logical_not(m0)
        c1 = plsc.cumsum(m1.astype(jnp.int32))
        pos1 = pB + c1 - 1
        plsc.store_scatter(bBs, [pos1], s, mask=m1)
        plsc.store_scatter(bBd, [pos1], d - HB, mask=m1)
        return pA + n0, pB + (L - n0)

    lax.fori_loop(0, EPW // L, step,
                  (jnp.zeros((L,), jnp.int32), jnp.zeros((L,), jnp.int32)))

    out = pl.ds(wid * CAP, CAP)
    pltpu.sync_copy(bAs, oAs.at[out])
    pltpu.sync_copy(bAd, oAd.at[out])
    pltpu.sync_copy(bBs, oBs.at[out])
    pltpu.sync_copy(bBd, oBd.at[out])


def _agg_body(t_hbm, srcA, dstA, srcB, dstB, zeros_hbm, out,
              src_v, dst_v, r0, r1, r2, r3, acc,
              g0, g1, g2, g3, s0, s1, s2, s3):
    cid = lax.axis_index("c")
    sid = lax.axis_index("s")
    bufs = (r0, r1, r2, r3)
    gsem = (g0, g1, g2, g3)
    ssem = (s0, s1, s2, s3)
    NBUF = 4
    NC2 = 2 * NCB
    ROUNDS = NC2 // NBUF

    @pl.when(cid == 0)
    def _():
        pltpu.sync_copy(srcA.at[2 * sid], src_v.at[pl.ds(0, NCB)])
        pltpu.sync_copy(srcA.at[2 * sid + 1], src_v.at[pl.ds(NCB, NCB)])
        pltpu.sync_copy(dstA.at[2 * sid], dst_v.at[pl.ds(0, NCB)])
        pltpu.sync_copy(dstA.at[2 * sid + 1], dst_v.at[pl.ds(NCB, NCB)])

    @pl.when(cid == 1)
    def _():
        pltpu.sync_copy(srcB.at[2 * sid], src_v.at[pl.ds(0, NCB)])
        pltpu.sync_copy(srcB.at[2 * sid + 1], src_v.at[pl.ds(NCB, NCB)])
        pltpu.sync_copy(dstB.at[2 * sid], dst_v.at[pl.ds(0, NCB)])
        pltpu.sync_copy(dstB.at[2 * sid + 1], dst_v.at[pl.ds(NCB, NCB)])

    for q in range(RPA // ZR):
        pltpu.sync_copy(zeros_hbm, acc.at[pl.ds(sid * RPA + q * ZR, ZR)])

    @pl.when(sid == NS - 1)
    def _():
        pltpu.sync_copy(zeros_hbm.at[pl.ds(0, TAILA)],
                        acc.at[pl.ds(TOFFA, TAILA)])

    plsc.subcore_barrier()

    for q in range(NBUF):
        pltpu.async_copy(t_hbm.at[src_v.at[q]], bufs[q], gsem[q])

    def rnd(jj, carry):
        j0 = jj * NBUF
        for q in range(NBUF):
            pltpu.make_async_copy(
                t_hbm.at[src_v.at[j0 + q]], bufs[q], gsem[q]).wait()
            pltpu.async_copy(bufs[q], acc.at[dst_v.at[j0 + q]], ssem[q],
                             add=True)
        for q in range(NBUF):
            pltpu.make_async_copy(
                bufs[q], acc.at[dst_v.at[j0 + q]], ssem[q]).wait()

            @pl.when(j0 + NBUF + q < NC2)
            def _():
                pltpu.async_copy(
                    t_hbm.at[src_v.at[j0 + NBUF + q]], bufs[q], gsem[q])
        return carry

    lax.fori_loop(0, ROUNDS, rnd, 0)

    plsc.subcore_barrier()
    sl = pl.ds(sid * RPA, RPA)
    tsl = pl.ds(TOFFA, HB - TOFFA)

    @pl.when(cid == 0)
    def _():
        pltpu.sync_copy(acc.at[sl], out.at[pl.ds(sid * RPA, RPA)])

        @pl.when(sid == NS - 1)
        def _():
            pltpu.sync_copy(acc.at[tsl], out.at[pl.ds(TOFFA, HB - TOFFA)])

    @pl.when(cid == 1)
    def _():
        pltpu.sync_copy(acc.at[sl], out.at[pl.ds(HB + sid * RPA, RPA)])

        @pl.when(sid == NS - 1)
        def _():
            pltpu.sync_copy(acc.at[tsl],
                            out.at[pl.ds(HB + TOFFA, HB - TOFFA)])


@functools.lru_cache(maxsize=None)
def _sc_kernels():
    mesh = plsc.VectorSubcoreMesh(
        core_axis_name="c", subcore_axis_name="s", num_cores=NC,
        num_subcores=NS)
    deg = functools.partial(
        pl.kernel,
        out_type=(
            jax.ShapeDtypeStruct((N, 1), jnp.float32),
            jax.ShapeDtypeStruct((N, 1), jnp.float32),
        ),
        mesh=mesh,
        scratch_types=(
            pltpu.VMEM((NCHD, CHD), jnp.int32),
            pltpu.VMEM((CHD, 1), jnp.float32),
            pltpu.VMEM_SHARED((N, 1), jnp.float32),
        ),
    )(_deg_body)
    bink = functools.partial(
        pl.kernel,
        out_type=tuple(
            jax.ShapeDtypeStruct((NW * CAP,), jnp.int32) for _ in range(4)
        ),
        mesh=mesh,
        scratch_types=(
            pltpu.VMEM((EPW,), jnp.int32),
            pltpu.VMEM((EPW,), jnp.int32),
            pltpu.VMEM((CAP,), jnp.int32),
            pltpu.VMEM((CAP,), jnp.int32),
            pltpu.VMEM((CAP,), jnp.int32),
            pltpu.VMEM((CAP,), jnp.int32),
        ),
        compiler_params=pltpu.CompilerParams(needs_layout_passes=False),
    )(_bin_body)
    agg = functools.partial(
        pl.kernel,
        out_type=jax.ShapeDtypeStruct((N, D), jnp.float32),
        mesh=mesh,
        scratch_types=(
            pltpu.VMEM((2 * NCB, CB), jnp.int32),
            pltpu.VMEM((2 * NCB, CB), jnp.int32),
            pltpu.VMEM((CB, D), jnp.float32),
            pltpu.VMEM((CB, D), jnp.float32),
            pltpu.VMEM((CB, D), jnp.float32),
            pltpu.VMEM((CB, D), jnp.float32),
            pltpu.VMEM_SHARED((ACCR, D), jnp.float32),
            pltpu.SemaphoreType.DMA,
            pltpu.SemaphoreType.DMA,
            pltpu.SemaphoreType.DMA,
            pltpu.SemaphoreType.DMA,
            pltpu.SemaphoreType.DMA,
            pltpu.SemaphoreType.DMA,
            pltpu.SemaphoreType.DMA,
            pltpu.SemaphoreType.DMA,
        ),
    )(_agg_body)
    return deg, bink, agg


BM = 2000
GRID = N // BM


def _dinv(d0, d1):
    deg = d0 + d1
    return jnp.where(deg > 0.0, lax.rsqrt(deg), 0.0)


def _lin_body(d0_ref, d1_ref, x_ref, w_ref, b_ref, o_ref):
    dinv = _dinv(d0_ref[...], d1_ref[...])
    h = jnp.dot(x_ref[...], w_ref[...], preferred_element_type=jnp.float32)
    o_ref[...] = (h + b_ref[...]) * dinv


def _mid_body(d0_ref, d1_ref, a_ref, g_ref, be_ref, w_ref, b_ref, o_ref):
    dinv = _dinv(d0_ref[...], d1_ref[...])
    z = a_ref[...] * dinv
    gs = g_ref[...] * lax.rsqrt(jnp.float32(1.0 + BN_EPS))
    z = jnp.maximum(z * gs + be_ref[...], 0.0)
    h = jnp.dot(z, w_ref[...], preferred_element_type=jnp.float32)
    o_ref[...] = (h + b_ref[...]) * dinv


def _lsm_body(d0_ref, d1_ref, a_ref, o_ref):
    dinv = _dinv(d0_ref[...], d1_ref[...])
    z = a_ref[...] * dinv
    m = jnp.max(z, axis=-1, keepdims=True)
    s = z - m
    o_ref[...] = s - jnp.log(jnp.sum(jnp.exp(s), axis=-1, keepdims=True))


_col = pl.BlockSpec((BM, 1), lambda i: (i, 0))
_row = pl.BlockSpec((BM, D), lambda i: (i, 0))
_vec = pl.BlockSpec((1, D), lambda i: (0, 0))
_mat = pl.BlockSpec((D, D), lambda i: (0, 0))
_out = jax.ShapeDtypeStruct((N, D), jnp.float32)

_lin = pl.pallas_call(
    _lin_body, grid=(GRID,),
    in_specs=[_col, _col, _row, _mat, _vec],
    out_specs=_row, out_shape=_out)

_mid = pl.pallas_call(
    _mid_body, grid=(GRID,),
    in_specs=[_col, _col, _row, _vec, _vec, _mat, _vec],
    out_specs=_row, out_shape=_out)

_lsm = pl.pallas_call(
    _lsm_body, grid=(GRID,),
    in_specs=[_col, _col, _row],
    out_specs=_row, out_shape=_out)


def kernel(x, adj_t, W1, b1, g1, be1, W2, b2, g2, be2, W3, b3):
    src = adj_t[0]
    dst = adj_t[1]
    dst3 = dst.reshape(NW, NCHD, CHD)
    ones_c = jnp.ones((CHD, 1), jnp.float32)
    zeros1 = jnp.zeros((RPT, 1), jnp.float32)
    zeros_t = jnp.zeros((ZR, D), jnp.float32)
    b1r, b2r, b3r = (v.reshape(1, D) for v in (b1, b2, b3))
    g1r, g2r = g1.reshape(1, D), g2.reshape(1, D)
    be1r, be2r = be1.reshape(1, D), be2.reshape(1, D)

    _deg, _bin, _agg = _sc_kernels()
    d0, d1 = _deg(dst3, ones_c, zeros1)
    sAf, dAf, sBf, dBf = _bin(src, dst)
    sA3 = sAf.reshape(NW, NCB, CB)
    dA3 = dAf.reshape(NW, NCB, CB)
    sB3 = sBf.reshape(NW, NCB, CB)
    dB3 = dBf.reshape(NW, NCB, CB)

    t1 = _lin(d0, d1, x, W1, b1r)
    a = _agg(t1, sA3, dA3, sB3, dB3, zeros_t)
    t2 = _mid(d0, d1, a, g1r, be1r, W2, b2r)
    a = _agg(t2, sA3, dA3, sB3, dB3, zeros_t)
    t3 = _mid(d0, d1, a, g2r, be2r, W3, b3r)
    a = _agg(t3, sA3, dA3, sB3, dB3, zeros_t)
    return _lsm(d0, d1, a)


# speedup vs baseline: 1.1976x; 1.0081x over previous
import functools

import jax
import jax.numpy as jnp
from jax import lax
from jax.experimental import pallas as pl
from jax.experimental.pallas import tpu as pltpu
from jax.experimental.pallas import tpu_sc as plsc

N = 10000
E = 320000
D = 128
BN_EPS = 1e-5

NC = 2
NS = 16
NW = NC * NS
EPW = E // NW
L = 16

CHD = 125
NCHD = EPW // CHD
RPT = 624
TAIL = N - NS * RPT
TOFF = NS * RPT

NBIN = 2
HB = N // NBIN
DUM = 8
ACCR = HB + DUM
CB = 128
NCB = 42
CAP = NCB * CB
ZR = 104
RPA = 312
TAILA = ACCR - NS * RPA
TOFFA = NS * RPA


def _deg_body(dst_hbm, ones_hbm, zeros1_hbm, deg0, deg1, dst_v, ones_v, dacc,
              dsem):
    cid = lax.axis_index("c")
    sid = lax.axis_index("s")
    wid = cid * NS + sid
    pltpu.sync_copy(dst_hbm.at[wid], dst_v)
    pltpu.sync_copy(ones_hbm, ones_v)
    pltpu.sync_copy(zeros1_hbm, dacc.at[pl.ds(sid * RPT, RPT)])

    @pl.when(sid == NS - 1)
    def _():
        pltpu.sync_copy(zeros1_hbm.at[pl.ds(0, TAIL)],
                        dacc.at[pl.ds(TOFF, TAIL)])

    plsc.subcore_barrier()

    def step(j, carry):
        @pl.when(j >= 4)
        def _():
            pltpu.make_async_copy(ones_v, dacc.at[dst_v.at[j - 4]],
                                  dsem).wait()

        pltpu.async_copy(ones_v, dacc.at[dst_v.at[j]], dsem, add=True)
        return carry

    lax.fori_loop(0, NCHD, step, 0)

    def drain(j, carry):
        pltpu.make_async_copy(ones_v, dacc.at[dst_v.at[j]], dsem).wait()
        return carry

    lax.fori_loop(NCHD - 4, NCHD, drain, 0)
    plsc.subcore_barrier()
    sl = pl.ds(sid * RPT, RPT)
    tl = pl.ds(TOFF, TAIL)

    @pl.when(cid == 0)
    def _():
        pltpu.sync_copy(dacc.at[sl], deg0.at[sl])

        @pl.when(sid == NS - 1)
        def _():
            pltpu.sync_copy(dacc.at[tl], deg0.at[tl])

    @pl.when(cid == 1)
    def _():
        pltpu.sync_copy(dacc.at[sl], deg1.at[sl])

        @pl.when(sid == NS - 1)
        def _():
            pltpu.sync_copy(dacc.at[tl], deg1.at[tl])


def _bin_body(src_hbm, dst_hbm, oAs, oAd, oBs, oBd,
              stage_s, stage_d, bAs, bAd, bBs, bBd):
    cid = lax.axis_index("c")
    sid = lax.axis_index("s")
    wid = cid * NS + sid
    pltpu.sync_copy(src_hbm.at[pl.ds(wid * EPW, EPW)], stage_s)
    pltpu.sync_copy(dst_hbm.at[pl.ds(wid * EPW, EPW)], stage_d)

    iota = lax.iota(jnp.int32, L)
    pad_d = HB + (iota % DUM)
    pad_s = wid * 256 + iota * 16

    def prefill(q, carry):
        sl = pl.ds(q * L, L)
        bAs[sl] = pad_s
        bAd[sl] = pad_d
        bBs[sl] = pad_s
        bBd[sl] = pad_d
        return carry

    lax.fori_loop(0, CAP // L, prefill, 0)

    def step(i, carry):
        pA, pB = carry
        sl = pl.ds(i * L, L)
        s = stage_s[sl]
        d = stage_d[sl]
        m0 = d < HB
        n0 = plsc.all_reduce_population_count(m0)
        c = plsc.cumsum(m0.astype(jnp.int32))
        pos0 = pA + c - 1
        plsc.store_scatter(bAs, [pos0], s, mask=m0)
        plsc.store_scatter(bAd, [pos0], d, mask=m0)
        m1 = jnp.logical_not(m0)
        c1 = plsc.cumsum(m1.astype(jnp.int32))
        pos1 = pB + c1 - 1
        plsc.store_scatter(bBs, [pos1], s, mask=m1)
        plsc.store_scatter(bBd, [pos1], d - HB, mask=m1)
        return pA + n0, pB + (L - n0)

    lax.fori_loop(0, EPW // L, step,
                  (jnp.zeros((L,), jnp.int32), jnp.zeros((L,), jnp.int32)))

    out = pl.ds(wid * CAP, CAP)
    pltpu.sync_copy(bAs, oAs.at[out])
    pltpu.sync_copy(bAd, oAd.at[out])
    pltpu.sync_copy(bBs, oBs.at[out])
    pltpu.sync_copy(bBd, oBd.at[out])


def _agg_body(t_hbm, srcA, dstA, srcB, dstB, zeros_hbm, out,
              src_v, dst_v, r0, r1, r2, r3, acc,
              g0, g1, g2, g3, s0, s1, s2, s3):
    cid = lax.axis_index("c")
    sid = lax.axis_index("s")
    bufs = (r0, r1, r2, r3)
    gsem = (g0, g1, g2, g3)
    ssem = (s0, s1, s2, s3)
    NBUF = 4
    NC2 = 2 * NCB
    ROUNDS = NC2 // NBUF

    @pl.when(cid == 0)
    def _():
        pltpu.sync_copy(srcA.at[2 * sid], src_v.at[pl.ds(0, NCB)])
        pltpu.sync_copy(srcA.at[2 * sid + 1], src_v.at[pl.ds(NCB, NCB)])
        pltpu.sync_copy(dstA.at[2 * sid], dst_v.at[pl.ds(0, NCB)])
        pltpu.sync_copy(dstA.at[2 * sid + 1], dst_v.at[pl.ds(NCB, NCB)])

    @pl.when(cid == 1)
    def _():
        pltpu.sync_copy(srcB.at[2 * sid], src_v.at[pl.ds(0, NCB)])
        pltpu.sync_copy(srcB.at[2 * sid + 1], src_v.at[pl.ds(NCB, NCB)])
        pltpu.sync_copy(dstB.at[2 * sid], dst_v.at[pl.ds(0, NCB)])
        pltpu.sync_copy(dstB.at[2 * sid + 1], dst_v.at[pl.ds(NCB, NCB)])

    for q in range(RPA // ZR):
        pltpu.sync_copy(zeros_hbm, acc.at[pl.ds(sid * RPA + q * ZR, ZR)])

    @pl.when(sid == NS - 1)
    def _():
        pltpu.sync_copy(zeros_hbm.at[pl.ds(0, TAILA)],
                        acc.at[pl.ds(TOFFA, TAILA)])

    plsc.subcore_barrier()

    for q in range(NBUF):
        pltpu.async_copy(t_hbm.at[src_v.at[q]], bufs[q], gsem[q])

    def rnd(jj, carry):
        j0 = jj * NBUF
        for q in range(NBUF):
            pltpu.make_async_copy(
                t_hbm.at[src_v.at[j0 + q]], bufs[q], gsem[q]).wait()
            pltpu.async_copy(bufs[q], acc.at[dst_v.at[j0 + q]], ssem[q],
                             add=True)
        for q in range(NBUF):
            pltpu.make_async_copy(
                bufs[q], acc.at[dst_v.at[j0 + q]], ssem[q]).wait()

            @pl.when(j0 + NBUF + q < NC2)
            def _():
                pltpu.async_copy(
                    t_hbm.at[src_v.at[j0 + NBUF + q]], bufs[q], gsem[q])
        return carry

    lax.fori_loop(0, ROUNDS, rnd, 0)

    plsc.subcore_barrier()
    sl = pl.ds(sid * RPA, RPA)
    tsl = pl.ds(TOFFA, HB - TOFFA)

    @pl.when(cid == 0)
    def _():
        pltpu.sync_copy(acc.at[sl], out.at[pl.ds(sid * RPA, RPA)])

        @pl.when(sid == NS - 1)
        def _():
            pltpu.sync_copy(acc.at[tsl], out.at[pl.ds(TOFFA, HB - TOFFA)])

    @pl.when(cid == 1)
    def _():
        pltpu.sync_copy(acc.at[sl], out.at[pl.ds(HB + sid * RPA, RPA)])

        @pl.when(sid == NS - 1)
        def _():
            pltpu.sync_copy(acc.at[tsl],
                            out.at[pl.ds(HB + TOFFA, HB - TOFFA)])


@functools.lru_cache(maxsize=None)
def _sc_kernels():
    mesh = plsc.VectorSubcoreMesh(
        core_axis_name="c", subcore_axis_name="s", num_cores=NC,
        num_subcores=NS)
    deg = functools.partial(
        pl.kernel,
        out_type=(
            jax.ShapeDtypeStruct((N, 1), jnp.float32),
            jax.ShapeDtypeStruct((N, 1), jnp.float32),
        ),
        mesh=mesh,
        scratch_types=(
            pltpu.VMEM((NCHD, CHD), jnp.int32),
            pltpu.VMEM((CHD, 1), jnp.float32),
            pltpu.VMEM_SHARED((N, 1), jnp.float32),
            pltpu.SemaphoreType.DMA,
        ),
    )(_deg_body)
    bink = functools.partial(
        pl.kernel,
        out_type=tuple(
            jax.ShapeDtypeStruct((NW * CAP,), jnp.int32) for _ in range(4)
        ),
        mesh=mesh,
        scratch_types=(
            pltpu.VMEM((EPW,), jnp.int32),
            pltpu.VMEM((EPW,), jnp.int32),
            pltpu.VMEM((CAP,), jnp.int32),
            pltpu.VMEM((CAP,), jnp.int32),
            pltpu.VMEM((CAP,), jnp.int32),
            pltpu.VMEM((CAP,), jnp.int32),
        ),
        compiler_params=pltpu.CompilerParams(needs_layout_passes=False),
    )(_bin_body)
    agg = functools.partial(
        pl.kernel,
        out_type=jax.ShapeDtypeStruct((N, D), jnp.float32),
        mesh=mesh,
        scratch_types=(
            pltpu.VMEM((2 * NCB, CB), jnp.int32),
            pltpu.VMEM((2 * NCB, CB), jnp.int32),
            pltpu.VMEM((CB, D), jnp.float32),
            pltpu.VMEM((CB, D), jnp.float32),
            pltpu.VMEM((CB, D), jnp.float32),
            pltpu.VMEM((CB, D), jnp.float32),
            pltpu.VMEM_SHARED((ACCR, D), jnp.float32),
            pltpu.SemaphoreType.DMA,
            pltpu.SemaphoreType.DMA,
            pltpu.SemaphoreType.DMA,
            pltpu.SemaphoreType.DMA,
            pltpu.SemaphoreType.DMA,
            pltpu.SemaphoreType.DMA,
            pltpu.SemaphoreType.DMA,
            pltpu.SemaphoreType.DMA,
        ),
    )(_agg_body)
    return deg, bink, agg


BM = 2000
GRID = N // BM


def _dinv(d0, d1):
    deg = d0 + d1
    return jnp.where(deg > 0.0, lax.rsqrt(deg), 0.0)


def _lin_body(d0_ref, d1_ref, x_ref, w_ref, b_ref, o_ref):
    dinv = _dinv(d0_ref[...], d1_ref[...])
    h = jnp.dot(x_ref[...], w_ref[...], preferred_element_type=jnp.float32)
    o_ref[...] = (h + b_ref[...]) * dinv


def _mid_body(d0_ref, d1_ref, a_ref, g_ref, be_ref, w_ref, b_ref, o_ref):
    dinv = _dinv(d0_ref[...], d1_ref[...])
    z = a_ref[...] * dinv
    gs = g_ref[...] * lax.rsqrt(jnp.float32(1.0 + BN_EPS))
    z = jnp.maximum(z * gs + be_ref[...], 0.0)
    h = jnp.dot(z, w_ref[...], preferred_element_type=jnp.float32)
    o_ref[...] = (h + b_ref[...]) * dinv


def _lsm_body(d0_ref, d1_ref, a_ref, o_ref):
    dinv = _dinv(d0_ref[...], d1_ref[...])
    z = a_ref[...] * dinv
    m = jnp.max(z, axis=-1, keepdims=True)
    s = z - m
    o_ref[...] = s - jnp.log(jnp.sum(jnp.exp(s), axis=-1, keepdims=True))


_col = pl.BlockSpec((BM, 1), lambda i: (i, 0))
_row = pl.BlockSpec((BM, D), lambda i: (i, 0))
_vec = pl.BlockSpec((1, D), lambda i: (0, 0))
_mat = pl.BlockSpec((D, D), lambda i: (0, 0))
_out = jax.ShapeDtypeStruct((N, D), jnp.float32)

_lin = pl.pallas_call(
    _lin_body, grid=(GRID,),
    in_specs=[_col, _col, _row, _mat, _vec],
    out_specs=_row, out_shape=_out)

_mid = pl.pallas_call(
    _mid_body, grid=(GRID,),
    in_specs=[_col, _col, _row, _vec, _vec, _mat, _vec],
    out_specs=_row, out_shape=_out)

_lsm = pl.pallas_call(
    _lsm_body, grid=(GRID,),
    in_specs=[_col, _col, _row],
    out_specs=_row, out_shape=_out)


def kernel(x, adj_t, W1, b1, g1, be1, W2, b2, g2, be2, W3, b3):
    src = adj_t[0]
    dst = adj_t[1]
    dst3 = dst.reshape(NW, NCHD, CHD)
    ones_c = jnp.ones((CHD, 1), jnp.float32)
    zeros1 = jnp.zeros((RPT, 1), jnp.float32)
    zeros_t = jnp.zeros((ZR, D), jnp.float32)
    b1r, b2r, b3r = (v.reshape(1, D) for v in (b1, b2, b3))
    g1r, g2r = g1.reshape(1, D), g2.reshape(1, D)
    be1r, be2r = be1.reshape(1, D), be2.reshape(1, D)

    _deg, _bin, _agg = _sc_kernels()
    d0, d1 = _deg(dst3, ones_c, zeros1)
    sAf, dAf, sBf, dBf = _bin(src, dst)
    sA3 = sAf.reshape(NW, NCB, CB)
    dA3 = dAf.reshape(NW, NCB, CB)
    sB3 = sBf.reshape(NW, NCB, CB)
    dB3 = dBf.reshape(NW, NCB, CB)

    t1 = _lin(d0, d1, x, W1, b1r)
    a = _agg(t1, sA3, dA3, sB3, dB3, zeros_t)
    t2 = _mid(d0, d1, a, g1r, be1r, W2, b2r)
    a = _agg(t2, sA3, dA3, sB3, dB3, zeros_t)
    t3 = _mid(d0, d1, a, g2r, be2r, W3, b3r)
    a = _agg(t3, sA3, dA3, sB3, dB3, zeros_t)
    return _lsm(d0, d1, a)
